# baseline (speedup 1.0000x reference)
import functools

import jax
import jax.numpy as jnp
from jax.experimental import pallas as pl
from jax.experimental.pallas import tpu as pltpu

B, D, K = 4096, 2048, 8
BM = 256
NBLK = B // BM


def _encode_block(idx_ref, keep_ref, mask_ref, x_hbm, w_ref, b_ref, out_ref,
                  xtile, sem):
    i = pl.program_id(0)
    idx = idx_ref[0]

    def fetch(block, slot):
        def issue(r, c):
            @pl.when(keep_ref[block * BM + r] != 0)
            def _():
                pltpu.make_async_copy(
                    x_hbm.at[pl.ds(block * BM + r, 1), :],
                    xtile.at[slot, pl.ds(r, 1), :],
                    sem.at[slot]).start()
            return c
        jax.lax.fori_loop(0, BM, issue, 0, unroll=8)

    def drain(block, slot):
        def wait(r, c):
            @pl.when(keep_ref[block * BM + r] != 0)
            def _():
                pltpu.make_async_copy(
                    x_hbm.at[pl.ds(block * BM + r, 1), :],
                    xtile.at[slot, pl.ds(r, 1), :],
                    sem.at[slot]).wait()
            return c
        jax.lax.fori_loop(0, BM, wait, 0, unroll=8)

    slot = jax.lax.rem(i, 2)

    @pl.when(i == 0)
    def _():
        fetch(0, 0)

    @pl.when(i + 1 < NBLK)
    def _():
        fetch(i + 1, jax.lax.rem(i + 1, 2))

    drain(i, slot)

    onehot = (jax.lax.broadcasted_iota(jnp.int32, (1, K), 1) == idx)
    sel = jnp.sum(mask_ref[...] * onehot.astype(jnp.float32), axis=1,
                  keepdims=True)
    keep = sel > 0.5
    acc = jax.lax.dot_general(
        xtile[slot], w_ref[...], (((1,), (1,)), ((), ())),
        preferred_element_type=jnp.float32)
    acc = acc + b_ref[...]
    out_ref[...] = jnp.where(keep, acc, 0.0)


def kernel(input_data, selection_mask, W, bvec, modality_idx):
    idx = jnp.atleast_1d(jnp.asarray(modality_idx, dtype=jnp.int32))
    sel_col = jnp.take_along_axis(
        selection_mask, jnp.reshape(idx, (1, 1)), axis=1)[:, 0]
    keep_i32 = (sel_col > 0.5).astype(jnp.int32)
    grid_spec = pltpu.PrefetchScalarGridSpec(
        num_scalar_prefetch=2,
        grid=(NBLK,),
        in_specs=[
            pl.BlockSpec((BM, K), lambda i, *_: (i, 0)),
            pl.BlockSpec(memory_space=pl.ANY),
            pl.BlockSpec((D, D), lambda i, *_: (0, 0)),
            pl.BlockSpec((1, D), lambda i, *_: (0, 0)),
        ],
        out_specs=pl.BlockSpec((BM, D), lambda i, *_: (i, 0)),
        scratch_shapes=[
            pltpu.VMEM((2, BM, D), jnp.float32),
            pltpu.SemaphoreType.DMA((2,)),
        ],
    )
    return pl.pallas_call(
        _encode_block,
        grid_spec=grid_spec,
        out_shape=jax.ShapeDtypeStruct((B, D), jnp.float32),
    )(idx, keep_i32, selection_mask, input_data, W, bvec.reshape(1, D))


# speedup vs baseline: 1.9862x; 1.9862x over previous
import jax
import jax.numpy as jnp
from jax.experimental import pallas as pl
from jax.experimental.pallas import tpu as pltpu

B, D, K = 4096, 2048, 8
BM = 256


def _encode_block(idx_ref, mask_ref, x_ref, w_ref, b_ref, out_ref):
    idx = idx_ref[0]
    onehot = (jax.lax.broadcasted_iota(jnp.int32, (1, K), 1) == idx)
    sel = jnp.sum(mask_ref[...] * onehot.astype(jnp.float32), axis=1,
                  keepdims=True)
    keep = sel > 0.5
    xb = x_ref[...].astype(jnp.bfloat16)
    wb = w_ref[...].astype(jnp.bfloat16)
    acc = jax.lax.dot_general(
        xb, wb, (((1,), (1,)), ((), ())),
        preferred_element_type=jnp.float32)
    acc = acc + b_ref[...]
    out_ref[...] = jnp.where(keep, acc, 0.0)


def kernel(input_data, selection_mask, W, bvec, modality_idx):
    idx = jnp.atleast_1d(jnp.asarray(modality_idx, dtype=jnp.int32))
    grid_spec = pltpu.PrefetchScalarGridSpec(
        num_scalar_prefetch=1,
        grid=(B // BM,),
        in_specs=[
            pl.BlockSpec((BM, K), lambda i, *_: (i, 0)),
            pl.BlockSpec((BM, D), lambda i, *_: (i, 0)),
            pl.BlockSpec((D, D), lambda i, *_: (0, 0)),
            pl.BlockSpec((1, D), lambda i, *_: (0, 0)),
        ],
        out_specs=pl.BlockSpec((BM, D), lambda i, *_: (i, 0)),
    )
    return pl.pallas_call(
        _encode_block,
        grid_spec=grid_spec,
        out_shape=jax.ShapeDtypeStruct((B, D), jnp.float32),
    )(idx, selection_mask, input_data, W, bvec.reshape(1, D))
